# baseline (speedup 1.0000x reference)
import jax
import jax.numpy as jnp
from jax import lax
from jax.experimental import pallas as pl
from jax.experimental.pallas import tpu as pltpu
from jax.experimental.pallas import tpu_sc as plsc

_NC, _NS = 2, 16
_NW = _NC * _NS

_ENT_ROWS = 1000000
_CHUNK = 31232
_PIECE = 3904
_TAIL = _ENT_ROWS - _NW * _CHUNK


def _sc_copy_body(ent_in, rel_in, ent_out, rel_out, sem):
    wid = lax.axis_index("s") * _NC + lax.axis_index("c")
    base = wid * _CHUNK
    copies = []
    for j in range(_CHUNK // _PIECE):
        sl = pl.ds(base + j * _PIECE, _PIECE)
        copies.append(pltpu.async_copy(ent_in.at[sl], ent_out.at[sl], sem))
    for c in copies:
        c.wait()

    @pl.when(wid == _NW - 1)
    def _copy_tail():
        sl = pl.ds(_NW * _CHUNK, _TAIL)
        pltpu.async_copy(ent_in.at[sl], ent_out.at[sl], sem).wait()

    @pl.when(wid == 0)
    def _copy_rel():
        pltpu.async_copy(rel_in.at[...], rel_out.at[...], sem).wait()


def kernel(x_dict, edge_index, entity_emb, rel_emb):
    fn = pl.kernel(
        _sc_copy_body,
        out_type=(
            jax.ShapeDtypeStruct(entity_emb.shape, entity_emb.dtype),
            jax.ShapeDtypeStruct(rel_emb.shape, rel_emb.dtype),
        ),
        mesh=plsc.VectorSubcoreMesh(core_axis_name="c", subcore_axis_name="s"),
        scratch_types=[pltpu.SemaphoreType.DMA],
    )
    ent_out, rel_out = fn(entity_emb, rel_emb)
    return (ent_out, rel_out)


# speedup vs baseline: 15.3198x; 15.3198x over previous
import jax
import jax.numpy as jnp
from jax import lax
from jax.experimental import pallas as pl
from jax.experimental.pallas import tpu as pltpu
from jax.experimental.pallas import tpu_sc as plsc

_NC, _NS = 2, 16
_NW = _NC * _NS

_ENT_ROWS = 1000000
_PIECE = 504
_NPIECE = 62
_CHUNK = _PIECE * _NPIECE
_TAIL = _ENT_ROWS - _NW * _CHUNK

_REL_ROWS = 1000
_REL_SPLIT = (0, 496, 1000)


def _sc_copy_body(ent_in, rel_in, ent_out, rel_out, buf0, buf1, sem_in, sem_out):
    wid = lax.axis_index("s") * _NC + lax.axis_index("c")
    base = wid * _CHUNK
    bufs = (buf0, buf1)

    def ent_slice(j, rows):
        return pl.ds(base + j * _PIECE, rows)

    in_h = [None, None]
    out_h = [None, None]
    in_h[0] = pltpu.async_copy(
        ent_in.at[ent_slice(0, _PIECE)], bufs[0].at[pl.ds(0, _PIECE)], sem_in)
    for j in range(_NPIECE):
        b = j % 2
        in_h[b].wait()
        if j >= 1:
            out_h[1 - b].wait()
        if j + 1 < _NPIECE:
            in_h[1 - b] = pltpu.async_copy(
                ent_in.at[ent_slice(j + 1, _PIECE)],
                bufs[1 - b].at[pl.ds(0, _PIECE)], sem_in)
        out_h[b] = pltpu.async_copy(
            bufs[b].at[pl.ds(0, _PIECE)],
            ent_out.at[ent_slice(j, _PIECE)], sem_out)
    out_h[(_NPIECE - 1) % 2].wait()

    @pl.when(wid == _NW - 1)
    def _copy_tail():
        sl = pl.ds(_NW * _CHUNK, _TAIL)
        pltpu.async_copy(ent_in.at[sl], buf0.at[pl.ds(0, _TAIL)], sem_in).wait()
        pltpu.async_copy(buf0.at[pl.ds(0, _TAIL)], ent_out.at[sl], sem_out).wait()

    @pl.when(wid == 0)
    def _copy_rel():
        for k in range(2):
            lo, hi = _REL_SPLIT[k], _REL_SPLIT[k + 1]
            sl = pl.ds(lo, hi - lo)
            pltpu.async_copy(rel_in.at[sl], buf0.at[pl.ds(0, hi - lo)], sem_in).wait()
            pltpu.async_copy(buf0.at[pl.ds(0, hi - lo)], rel_out.at[sl], sem_out).wait()


def kernel(x_dict, edge_index, entity_emb, rel_emb):
    fn = pl.kernel(
        _sc_copy_body,
        out_type=(
            jax.ShapeDtypeStruct(entity_emb.shape, entity_emb.dtype),
            jax.ShapeDtypeStruct(rel_emb.shape, rel_emb.dtype),
        ),
        mesh=plsc.VectorSubcoreMesh(core_axis_name="c", subcore_axis_name="s"),
        scratch_types=[
            pltpu.VMEM((_PIECE, 64), jnp.float32),
            pltpu.VMEM((_PIECE, 64), jnp.float32),
            pltpu.SemaphoreType.DMA,
            pltpu.SemaphoreType.DMA,
        ],
    )
    ent_out, rel_out = fn(entity_emb, rel_emb)
    return (ent_out, rel_out)


# speedup vs baseline: 15.6186x; 1.0195x over previous
import jax
import jax.numpy as jnp
from jax.experimental import pallas as pl
from jax.experimental.pallas import tpu as pltpu

_ENT_ROWS = 1000000
_PIECE = 4000
_NPIECE = _ENT_ROWS // _PIECE
_DEPTH = 8


def _tc_ring_body(ent_in, rel_in, ent_out, rel_out, bufs, relbuf, sem_in, sem_out):
    def sl(j):
        return pl.ds(j * _PIECE, _PIECE)

    in_h = [None] * _DEPTH
    out_h = [None] * _DEPTH
    for b in range(_DEPTH):
        in_h[b] = pltpu.make_async_copy(
            ent_in.at[sl(b)], bufs.at[b], sem_in)
        in_h[b].start()
    for j in range(_NPIECE):
        b = j % _DEPTH
        in_h[b].wait()
        if j >= _DEPTH:
            pass
        out_h[b] = pltpu.make_async_copy(
            bufs.at[b], ent_out.at[sl(j)], sem_out)
        out_h[b].start()
        nxt = j + _DEPTH
        if nxt < _NPIECE:
            out_h[b].wait()
            in_h[b] = pltpu.make_async_copy(
                ent_in.at[sl(nxt)], bufs.at[b], sem_in)
            in_h[b].start()
    for j in range(_NPIECE - _DEPTH, _NPIECE):
        if j >= 0:
            out_h[j % _DEPTH].wait()

    rel_in_h = pltpu.make_async_copy(rel_in.at[...], relbuf, sem_in)
    rel_in_h.start()
    rel_in_h.wait()
    rel_out_h = pltpu.make_async_copy(relbuf, rel_out.at[...], sem_out)
    rel_out_h.start()
    rel_out_h.wait()


def kernel(x_dict, edge_index, entity_emb, rel_emb):
    ent_out, rel_out = pl.pallas_call(
        _tc_ring_body,
        out_shape=(
            jax.ShapeDtypeStruct(entity_emb.shape, entity_emb.dtype),
            jax.ShapeDtypeStruct(rel_emb.shape, rel_emb.dtype),
        ),
        in_specs=[
            pl.BlockSpec(memory_space=pl.ANY),
            pl.BlockSpec(memory_space=pl.ANY),
        ],
        out_specs=(
            pl.BlockSpec(memory_space=pl.ANY),
            pl.BlockSpec(memory_space=pl.ANY),
        ),
        scratch_shapes=[
            pltpu.VMEM((_DEPTH, _PIECE, 64), jnp.float32),
            pltpu.VMEM((1000, 64), jnp.float32),
            pltpu.SemaphoreType.DMA,
            pltpu.SemaphoreType.DMA,
        ],
    )(entity_emb, rel_emb)
    return (ent_out, rel_out)


# speedup vs baseline: 15.6186x; 1.0000x over previous
import jax
import jax.numpy as jnp
from jax.experimental import pallas as pl
from jax.experimental.pallas import tpu as pltpu

_ENT_ROWS = 1000000
_PIECE = 4000
_NPIECE = _ENT_ROWS // _PIECE
_DEPTH = 8


def _tc_ring_body(ent_in, rel_in, ent_out, rel_out, bufs, relbuf, sem_in, sem_out):
    def sl(j):
        return pl.ds(j * _PIECE, _PIECE)

    in_h = [None] * _DEPTH
    out_h = [None] * _DEPTH
    for b in range(_DEPTH):
        in_h[b] = pltpu.make_async_copy(
            ent_in.at[sl(b)], bufs.at[b], sem_in.at[b])
        in_h[b].start()
    for j in range(_NPIECE):
        b = j % _DEPTH
        in_h[b].wait()
        out_h[b] = pltpu.make_async_copy(
            bufs.at[b], ent_out.at[sl(j)], sem_out.at[b])
        out_h[b].start()
        nxt = j + _DEPTH
        if nxt < _NPIECE:
            out_h[b].wait()
            in_h[b] = pltpu.make_async_copy(
                ent_in.at[sl(nxt)], bufs.at[b], sem_in.at[b])
            in_h[b].start()
    for j in range(_NPIECE - _DEPTH, _NPIECE):
        if j >= 0:
            out_h[j % _DEPTH].wait()

    rel_in_h = pltpu.make_async_copy(rel_in.at[...], relbuf, sem_in.at[0])
    rel_in_h.start()
    rel_in_h.wait()
    rel_out_h = pltpu.make_async_copy(relbuf, rel_out.at[...], sem_out.at[0])
    rel_out_h.start()
    rel_out_h.wait()


def kernel(x_dict, edge_index, entity_emb, rel_emb):
    ent_out, rel_out = pl.pallas_call(
        _tc_ring_body,
        out_shape=(
            jax.ShapeDtypeStruct(entity_emb.shape, entity_emb.dtype),
            jax.ShapeDtypeStruct(rel_emb.shape, rel_emb.dtype),
        ),
        in_specs=[
            pl.BlockSpec(memory_space=pl.ANY),
            pl.BlockSpec(memory_space=pl.ANY),
        ],
        out_specs=(
            pl.BlockSpec(memory_space=pl.ANY),
            pl.BlockSpec(memory_space=pl.ANY),
        ),
        scratch_shapes=[
            pltpu.VMEM((_DEPTH, _PIECE, 64), jnp.float32),
            pltpu.VMEM((1000, 64), jnp.float32),
            pltpu.SemaphoreType.DMA((_DEPTH,)),
            pltpu.SemaphoreType.DMA((_DEPTH,)),
        ],
    )(entity_emb, rel_emb)
    return (ent_out, rel_out)


# speedup vs baseline: 16.1404x; 1.0334x over previous
import jax
import jax.numpy as jnp
from jax.experimental import pallas as pl
from jax.experimental.pallas import tpu as pltpu

_ENT_ROWS = 1000000
_PIECE = 15624
_NPIECE = _ENT_ROWS // _PIECE
_TAIL = _ENT_ROWS - _NPIECE * _PIECE
_DEPTH = 6


def _tc_ring_body(ent_in, rel_in, ent_out, rel_out, bufs, relbuf, sem_in, sem_out):
    def sl(j):
        return pl.ds(j * _PIECE, _PIECE)

    in_h = [None] * _DEPTH
    out_h = [None] * _DEPTH
    for b in range(_DEPTH):
        in_h[b] = pltpu.make_async_copy(ent_in.at[sl(b)], bufs.at[b], sem_in)
        in_h[b].start()
    for j in range(_NPIECE):
        b = j % _DEPTH
        in_h[b].wait()
        out_h[b] = pltpu.make_async_copy(bufs.at[b], ent_out.at[sl(j)], sem_out)
        out_h[b].start()
        nxt = j + _DEPTH
        if nxt < _NPIECE:
            out_h[b].wait()
            in_h[b] = pltpu.make_async_copy(ent_in.at[sl(nxt)], bufs.at[b], sem_in)
            in_h[b].start()
    for j in range(max(0, _NPIECE - _DEPTH), _NPIECE):
        out_h[j % _DEPTH].wait()

    tsl = pl.ds(_NPIECE * _PIECE, _TAIL)
    th = pltpu.make_async_copy(ent_in.at[tsl], bufs.at[0].at[pl.ds(0, _TAIL)], sem_in)
    th.start()
    th.wait()
    th2 = pltpu.make_async_copy(bufs.at[0].at[pl.ds(0, _TAIL)], ent_out.at[tsl], sem_out)
    th2.start()
    th2.wait()

    rel_in_h = pltpu.make_async_copy(rel_in.at[...], relbuf, sem_in)
    rel_in_h.start()
    rel_in_h.wait()
    rel_out_h = pltpu.make_async_copy(relbuf, rel_out.at[...], sem_out)
    rel_out_h.start()
    rel_out_h.wait()


def kernel(x_dict, edge_index, entity_emb, rel_emb):
    ent_out, rel_out = pl.pallas_call(
        _tc_ring_body,
        out_shape=(
            jax.ShapeDtypeStruct(entity_emb.shape, entity_emb.dtype),
            jax.ShapeDtypeStruct(rel_emb.shape, rel_emb.dtype),
        ),
        in_specs=[
            pl.BlockSpec(memory_space=pl.ANY),
            pl.BlockSpec(memory_space=pl.ANY),
        ],
        out_specs=(
            pl.BlockSpec(memory_space=pl.ANY),
            pl.BlockSpec(memory_space=pl.ANY),
        ),
        scratch_shapes=[
            pltpu.VMEM((_DEPTH, _PIECE, 64), jnp.float32),
            pltpu.VMEM((1000, 64), jnp.float32),
            pltpu.SemaphoreType.DMA,
            pltpu.SemaphoreType.DMA,
        ],
    )(entity_emb, rel_emb)
    return (ent_out, rel_out)
